# baseline (speedup 1.0000x reference)
import functools

import jax
import jax.numpy as jnp
from jax import lax
from jax.experimental import pallas as pl
from jax.experimental.pallas import tpu as pltpu
from jax.experimental.pallas import tpu_sc as plsc

N = 10000
E = 320000
D = 128
G = 8
EPS = 1e-5

NC = 2
NS = 16
NW = NC * NS
EPW = E // NW
CHUNK = 80
NCH = EPW // CHUNK
NSLOT = 3
NSEG = 5
CPS = NCH // NSEG
NPAD = 10240
RPT = NPAD // NS

_mesh = plsc.VectorSubcoreMesh(core_axis_name="c", subcore_axis_name="s")


@functools.partial(
    pl.kernel,
    out_type=jax.ShapeDtypeStruct((NW, N), jnp.float32),
    mesh=_mesh,
    scratch_types=[
        pltpu.VMEM((EPW,), jnp.int32),
        pltpu.VMEM((N,), jnp.float32),
    ],
    compiler_params=pltpu.CompilerParams(needs_layout_passes=False),
)
def _deg_kernel(dst_hbm, out_hbm, dsti_v, hist_v):
    wid = lax.axis_index("s") * NC + lax.axis_index("c")

    zeros16 = jnp.zeros((16,), jnp.float32)

    def zbody(i, carry):
        hist_v[pl.ds(i * 16, 16)] = zeros16
        return carry

    lax.fori_loop(0, N // 16, zbody, 0, unroll=4)

    pltpu.sync_copy(dst_hbm.at[wid], dsti_v)

    ones16 = jnp.ones((16,), jnp.float32)

    def body(i, carry):
        idx = dsti_v[pl.ds(i * 16, 16)]
        plsc.addupdate_scatter(hist_v, [idx], ones16)
        return carry

    lax.fori_loop(0, EPW // 16, body, 0, unroll=4)

    pltpu.sync_copy(hist_v, out_hbm.at[wid])


SEGE = CPS * CHUNK


@functools.partial(
    pl.kernel,
    out_type=jax.ShapeDtypeStruct((NC, NPAD, D), jnp.float32),
    mesh=_mesh,
    scratch_types=[
        pltpu.VMEM_SHARED((NPAD, D), jnp.float32),
        pltpu.VMEM((EPW,), jnp.int32),
        pltpu.VMEM((SEGE,), jnp.int32),
        pltpu.VMEM((CHUNK, D), jnp.float32),
        pltpu.VMEM((CHUNK, D), jnp.float32),
        pltpu.VMEM((CHUNK, D), jnp.float32),
        pltpu.SemaphoreType.DMA,
        pltpu.SemaphoreType.DMA,
        pltpu.SemaphoreType.DMA,
        pltpu.SemaphoreType.DMA,
        pltpu.SemaphoreType.DMA,
        pltpu.SemaphoreType.DMA,
    ],
)
def _scatter_kernel(hp_hbm, src_hbm, dst_hbm, out_hbm,
                    acc, srci_v, dsti_v, rows0, rows1, rows2,
                    semg0, semg1, semg2, sems0, sems1, sems2):
    cid = lax.axis_index("c")
    sid = lax.axis_index("s")
    wid = sid * NC + cid

    rows = (rows0, rows1, rows2)
    semg = (semg0, semg1, semg2)
    sems = (sems0, sems1, sems2)

    def gslice(c):
        return srci_v.at[pl.ds(c * CHUNK, CHUNK)]

    def dslice(j):
        return dsti_v.at[pl.ds(j * CHUNK, CHUNK)]

    pltpu.sync_copy(src_hbm.at[pl.ds(wid * EPW, EPW)], srci_v)

    zeros16 = jnp.zeros((16,), jnp.float32)

    def zb(i, carry):
        rows0[i // 8, pl.ds((i % 8) * 16, 16)] = zeros16
        return carry

    lax.fori_loop(0, 64 * 8, zb, 0, unroll=8)

    def zc(j, carry):
        pltpu.sync_copy(rows0.at[pl.ds(0, 64)],
                        acc.at[pl.ds(sid * RPT + j * 64, 64)])
        return carry

    lax.fori_loop(0, RPT // 64, zc, 0)
    plsc.subcore_barrier()

    for seg in range(NSEG):
        base_c = seg * CPS
        pltpu.sync_copy(dst_hbm.at[pl.ds(wid * EPW + seg * SEGE, SEGE)], dsti_v)
        pltpu.async_copy(hp_hbm.at[gslice(base_c)], rows0, semg0)

        def step(j, b):
            @pl.when(j >= 1)
            def _():
                pltpu.make_async_copy(rows[(b + 2) % 3], acc.at[dslice(0)],
                                      sems[(b + 2) % 3]).wait()

            @pl.when(j + 1 < CPS)
            def _():
                pltpu.async_copy(hp_hbm.at[gslice(base_c + j + 1)],
                                 rows[(b + 1) % 3], semg[(b + 1) % 3])

            pltpu.make_async_copy(hp_hbm.at[gslice(base_c + j)],
                                  rows[b], semg[b]).wait()
            pltpu.async_copy(rows[b], acc.at[dslice(j)], sems[b], add=True)

        def triple(t, carry):
            for k in range(3):
                step(t * 3 + k, k)
            return carry

        lax.fori_loop(0, CPS // 3, triple, 0)
        step(CPS - 1, (CPS - 1) % 3)

        x = (CPS - 1) % 3
        pltpu.make_async_copy(rows[x], acc.at[dslice(0)], sems[x]).wait()

    plsc.subcore_barrier()
    pltpu.sync_copy(acc.at[pl.ds(sid * RPT, RPT)],
                    out_hbm.at[cid, pl.ds(sid * RPT, RPT)])


def _dis_from(degT):
    return lax.rsqrt(jnp.sum(degT, axis=1, keepdims=True) + 2.0)


def _pre_body(x_ref, w_ref, degp_ref, out_ref):
    dis = _dis_from(degp_ref[...])
    out_ref[...] = dis * jnp.dot(x_ref[...], w_ref[...],
                                 preferred_element_type=jnp.float32)


def _mid_body(s_ref, hp_ref, degp_ref, b_ref, g_ref, bt_ref, hres_ref,
              wn_ref, h_out_ref, hpn_out_ref):
    dis = _dis_from(degp_ref[...])
    pre = dis * (s_ref[0, :N] + s_ref[1, :N] + 2.0 * hp_ref[...]) + b_ref[...]
    mu = jnp.mean(pre, axis=0, keepdims=True)
    var = jnp.mean((pre - mu) ** 2, axis=0, keepdims=True)
    bn = g_ref[...] * (pre - mu) * lax.rsqrt(var + EPS) + bt_ref[...]
    h_new = jnp.maximum(bn, 0.0) + hres_ref[...]
    h_out_ref[...] = h_new
    hpn_out_ref[...] = dis * jnp.dot(h_new, wn_ref[...],
                                     preferred_element_type=jnp.float32)


def _final_body(s_ref, hp_ref, degp_ref, b_ref, batch_ref, fcw_ref, fcb_ref,
                out_ref):
    dis = _dis_from(degp_ref[...])
    h3 = dis * (s_ref[0, :N] + s_ref[1, :N] + 2.0 * hp_ref[...]) + b_ref[...]
    gids = lax.broadcasted_iota(jnp.int32, (G, N), 0)
    onehot = (gids == batch_ref[...]).astype(jnp.float32)
    pooled = jnp.dot(onehot, h3, preferred_element_type=jnp.float32)
    res = jnp.dot(pooled, fcw_ref[...],
                  preferred_element_type=jnp.float32) + fcb_ref[...]
    out_ref[...] = jnp.broadcast_to(res, (G, 128))


_pre_call = pl.pallas_call(
    _pre_body, out_shape=jax.ShapeDtypeStruct((N, D), jnp.float32))

_mid_call = pl.pallas_call(
    _mid_body,
    out_shape=(jax.ShapeDtypeStruct((N, D), jnp.float32),
               jax.ShapeDtypeStruct((N, D), jnp.float32)))

_final_call = pl.pallas_call(
    _final_body, out_shape=jax.ShapeDtypeStruct((G, 128), jnp.float32))


def kernel(x, edge_index, batch, W1, b1, g1, bt1, W2, b2, g2, bt2, W3, b3,
           fcW, fcb):
    src = edge_index[0]
    dst = edge_index[1]

    degp = _deg_kernel(dst.reshape(NW, EPW)).T

    b1r = b1.reshape(1, D); g1r = g1.reshape(1, D); bt1r = bt1.reshape(1, D)
    b2r = b2.reshape(1, D); g2r = g2.reshape(1, D); bt2r = bt2.reshape(1, D)
    b3r = b3.reshape(1, D)
    batch_r = batch.reshape(1, N)
    fcb_r = fcb.reshape(1, 1)

    h1p = _pre_call(x, W1, degp)
    s1 = _scatter_kernel(h1p, src, dst)
    h_after1, h2p = _mid_call(s1, h1p, degp, b1r, g1r, bt1r, x, W2)
    s2 = _scatter_kernel(h2p, src, dst)
    h_after2, h3p = _mid_call(s2, h2p, degp, b2r, g2r, bt2r, h_after1, W3)
    s3 = _scatter_kernel(h3p, src, dst)
    out = _final_call(s3, h3p, degp, b3r, batch_r, fcW, fcb_r)
    return out[:, :1]


# speedup vs baseline: 1.0711x; 1.0711x over previous
import functools

import jax
import jax.numpy as jnp
from jax import lax
from jax.experimental import pallas as pl
from jax.experimental.pallas import tpu as pltpu
from jax.experimental.pallas import tpu_sc as plsc

N = 10000
E = 320000
D = 128
G = 8
EPS = 1e-5

NC = 2
NS = 16
NW = NC * NS
EPW = E // NW
CHUNK = 125
NCH = EPW // CHUNK
NSLOT = 2
NSEG = 5
CPS = NCH // NSEG
NPAD = 10240
RPT = NPAD // NS

_mesh = plsc.VectorSubcoreMesh(core_axis_name="c", subcore_axis_name="s")


@functools.partial(
    pl.kernel,
    out_type=jax.ShapeDtypeStruct((NW, N), jnp.float32),
    mesh=_mesh,
    scratch_types=[
        pltpu.VMEM((EPW,), jnp.int32),
        pltpu.VMEM((N,), jnp.float32),
    ],
    compiler_params=pltpu.CompilerParams(needs_layout_passes=False),
)
def _deg_kernel(dst_hbm, out_hbm, dsti_v, hist_v):
    wid = lax.axis_index("s") * NC + lax.axis_index("c")

    zeros16 = jnp.zeros((16,), jnp.float32)

    def zbody(i, carry):
        hist_v[pl.ds(i * 16, 16)] = zeros16
        return carry

    lax.fori_loop(0, N // 16, zbody, 0, unroll=4)

    pltpu.sync_copy(dst_hbm.at[wid], dsti_v)

    ones16 = jnp.ones((16,), jnp.float32)

    def body(i, carry):
        idx = dsti_v[pl.ds(i * 16, 16)]
        plsc.addupdate_scatter(hist_v, [idx], ones16)
        return carry

    lax.fori_loop(0, EPW // 16, body, 0, unroll=4)

    pltpu.sync_copy(hist_v, out_hbm.at[wid])


SEGE = CPS * CHUNK


@functools.partial(
    pl.kernel,
    out_type=jax.ShapeDtypeStruct((NC, NPAD, D), jnp.float32),
    mesh=_mesh,
    scratch_types=[
        pltpu.VMEM_SHARED((NPAD, D), jnp.float32),
        pltpu.VMEM((NCH, CHUNK), jnp.int32),
        pltpu.VMEM((CPS, CHUNK), jnp.int32),
        pltpu.VMEM((CHUNK, D), jnp.float32),
        pltpu.VMEM((CHUNK, D), jnp.float32),
        pltpu.SemaphoreType.DMA,
        pltpu.SemaphoreType.DMA,
        pltpu.SemaphoreType.DMA,
        pltpu.SemaphoreType.DMA,
    ],
)
def _scatter_kernel(hp_hbm, src_hbm, dst_hbm, out_hbm,
                    acc, srci_v, dsti_v, rows0, rows1,
                    semg0, semg1, sems0, sems1):
    cid = lax.axis_index("c")
    sid = lax.axis_index("s")
    wid = sid * NC + cid

    rows = (rows0, rows1)
    semg = (semg0, semg1)
    sems = (sems0, sems1)

    def gslice(c):
        return srci_v.at[c]

    def dslice(j):
        return dsti_v.at[j]

    pltpu.sync_copy(src_hbm.at[wid], srci_v)

    zeros16 = jnp.zeros((16,), jnp.float32)

    def zb(i, carry):
        rows0[i // 8, pl.ds((i % 8) * 16, 16)] = zeros16
        return carry

    lax.fori_loop(0, 64 * 8, zb, 0, unroll=8)

    def zc(j, carry):
        pltpu.sync_copy(rows0.at[pl.ds(0, 64)],
                        acc.at[pl.ds(sid * RPT + j * 64, 64)])
        return carry

    lax.fori_loop(0, RPT // 64, zc, 0)
    plsc.subcore_barrier()

    for seg in range(NSEG):
        base_c = seg * CPS
        pltpu.sync_copy(dst_hbm.at[wid, pl.ds(seg * CPS, CPS)], dsti_v)
        pltpu.async_copy(hp_hbm.at[gslice(base_c)], rows0, semg0)

        def step(j, b):
            nb = 1 - b

            @pl.when(j >= 1)
            def _():
                pltpu.make_async_copy(rows[nb], acc.at[dslice(0)],
                                      sems[nb]).wait()

            @pl.when(j + 1 < CPS)
            def _():
                pltpu.async_copy(hp_hbm.at[gslice(base_c + j + 1)],
                                 rows[nb], semg[nb])

            pltpu.make_async_copy(hp_hbm.at[gslice(base_c + j)],
                                  rows[b], semg[b]).wait()
            pltpu.async_copy(rows[b], acc.at[dslice(j)], sems[b], add=True)

        def pair(t, carry):
            for k in range(2):
                step(t * 2 + k, k)
            return carry

        lax.fori_loop(0, CPS // 2, pair, 0)

        x = (CPS - 1) % 2
        pltpu.make_async_copy(rows[x], acc.at[dslice(0)], sems[x]).wait()

    plsc.subcore_barrier()
    pltpu.sync_copy(acc.at[pl.ds(sid * RPT, RPT)],
                    out_hbm.at[cid, pl.ds(sid * RPT, RPT)])


def _dis_from(degT):
    return lax.rsqrt(jnp.sum(degT, axis=1, keepdims=True) + 2.0)


def _pre_body(x_ref, w_ref, degp_ref, out_ref):
    dis = _dis_from(degp_ref[...])
    out_ref[...] = dis * jnp.dot(x_ref[...], w_ref[...],
                                 preferred_element_type=jnp.float32)


def _mid_body(s_ref, hp_ref, degp_ref, b_ref, g_ref, bt_ref, hres_ref,
              wn_ref, h_out_ref, hpn_out_ref):
    dis = _dis_from(degp_ref[...])
    pre = dis * (s_ref[0, :N] + s_ref[1, :N] + 2.0 * hp_ref[...]) + b_ref[...]
    mu = jnp.mean(pre, axis=0, keepdims=True)
    var = jnp.mean((pre - mu) ** 2, axis=0, keepdims=True)
    bn = g_ref[...] * (pre - mu) * lax.rsqrt(var + EPS) + bt_ref[...]
    h_new = jnp.maximum(bn, 0.0) + hres_ref[...]
    h_out_ref[...] = h_new
    hpn_out_ref[...] = dis * jnp.dot(h_new, wn_ref[...],
                                     preferred_element_type=jnp.float32)


def _final_body(s_ref, hp_ref, degp_ref, b_ref, batch_ref, fcw_ref, fcb_ref,
                out_ref):
    dis = _dis_from(degp_ref[...])
    h3 = dis * (s_ref[0, :N] + s_ref[1, :N] + 2.0 * hp_ref[...]) + b_ref[...]
    gids = lax.broadcasted_iota(jnp.int32, (G, N), 0)
    onehot = (gids == batch_ref[...]).astype(jnp.float32)
    pooled = jnp.dot(onehot, h3, preferred_element_type=jnp.float32)
    res = jnp.dot(pooled, fcw_ref[...],
                  preferred_element_type=jnp.float32) + fcb_ref[...]
    out_ref[...] = jnp.broadcast_to(res, (G, 128))


_pre_call = pl.pallas_call(
    _pre_body, out_shape=jax.ShapeDtypeStruct((N, D), jnp.float32))

_mid_call = pl.pallas_call(
    _mid_body,
    out_shape=(jax.ShapeDtypeStruct((N, D), jnp.float32),
               jax.ShapeDtypeStruct((N, D), jnp.float32)))

_final_call = pl.pallas_call(
    _final_body, out_shape=jax.ShapeDtypeStruct((G, 128), jnp.float32))


def kernel(x, edge_index, batch, W1, b1, g1, bt1, W2, b2, g2, bt2, W3, b3,
           fcW, fcb):
    src = edge_index[0].reshape(NW, NCH, CHUNK)
    dst = edge_index[1].reshape(NW, NCH, CHUNK)

    degp = _deg_kernel(edge_index[1].reshape(NW, EPW)).T

    b1r = b1.reshape(1, D); g1r = g1.reshape(1, D); bt1r = bt1.reshape(1, D)
    b2r = b2.reshape(1, D); g2r = g2.reshape(1, D); bt2r = bt2.reshape(1, D)
    b3r = b3.reshape(1, D)
    batch_r = batch.reshape(1, N)
    fcb_r = fcb.reshape(1, 1)

    h1p = _pre_call(x, W1, degp)
    s1 = _scatter_kernel(h1p, src, dst)
    h_after1, h2p = _mid_call(s1, h1p, degp, b1r, g1r, bt1r, x, W2)
    s2 = _scatter_kernel(h2p, src, dst)
    h_after2, h3p = _mid_call(s2, h2p, degp, b2r, g2r, bt2r, h_after1, W3)
    s3 = _scatter_kernel(h3p, src, dst)
    out = _final_call(s3, h3p, degp, b3r, batch_r, fcW, fcb_r)
    return out[:, :1]


# speedup vs baseline: 1.1043x; 1.0310x over previous
import functools

import jax
import jax.numpy as jnp
from jax import lax
from jax.experimental import pallas as pl
from jax.experimental.pallas import tpu as pltpu
from jax.experimental.pallas import tpu_sc as plsc

N = 10000
E = 320000
D = 128
G = 8
EPS = 1e-5

NC = 2
NS = 16
NW = NC * NS
EPW = E // NW
CHUNK = 125
NCH = EPW // CHUNK
NSLOT = 2
NSEG = 5
CPS = NCH // NSEG
NPAD = 10240
RPT = NPAD // NS

_mesh = plsc.VectorSubcoreMesh(core_axis_name="c", subcore_axis_name="s")


@functools.partial(
    pl.kernel,
    out_type=jax.ShapeDtypeStruct((NW, N), jnp.float32),
    mesh=_mesh,
    scratch_types=[
        pltpu.VMEM((EPW,), jnp.int32),
        pltpu.VMEM((N,), jnp.float32),
    ],
    compiler_params=pltpu.CompilerParams(needs_layout_passes=False),
)
def _deg_kernel(dst_hbm, out_hbm, dsti_v, hist_v):
    wid = lax.axis_index("s") * NC + lax.axis_index("c")

    zeros16 = jnp.zeros((16,), jnp.float32)

    def zbody(i, carry):
        hist_v[pl.ds(i * 16, 16)] = zeros16
        return carry

    lax.fori_loop(0, N // 16, zbody, 0, unroll=4)

    pltpu.sync_copy(dst_hbm.at[wid], dsti_v)

    ones16 = jnp.ones((16,), jnp.float32)

    def body(i, carry):
        idx = dsti_v[pl.ds(i * 16, 16)]
        plsc.addupdate_scatter(hist_v, [idx], ones16)
        return carry

    lax.fori_loop(0, EPW // 16, body, 0, unroll=4)

    pltpu.sync_copy(hist_v, out_hbm.at[wid])


@functools.partial(
    pl.kernel,
    out_type=jax.ShapeDtypeStruct((NC, NPAD, D), jnp.float32),
    mesh=_mesh,
    scratch_types=[
        pltpu.VMEM_SHARED((NPAD, D), jnp.float32),
        pltpu.VMEM((NCH // 2, CHUNK), jnp.int32),
        pltpu.VMEM((NCH // 2, CHUNK), jnp.int32),
        pltpu.VMEM((CHUNK, D), jnp.float32),
        pltpu.VMEM((CHUNK, D), jnp.float32),
        pltpu.SemaphoreType.DMA,
        pltpu.SemaphoreType.DMA,
    ],
)
def _scatter_kernel(hp_hbm, src_hbm, dst_hbm, out_hbm,
                    acc, srci_v, dsti_v, rows0, rows1, sem0, sem1):
    cid = lax.axis_index("c")
    sid = lax.axis_index("s")
    wid = sid * NC + cid
    half_n = NCH // 2

    zeros16 = jnp.zeros((16,), jnp.float32)

    def zb(i, carry):
        rows0[i // 8, pl.ds((i % 8) * 16, 16)] = zeros16
        return carry

    lax.fori_loop(0, 64 * 8, zb, 0, unroll=8)

    def zc(j, carry):
        pltpu.sync_copy(rows0.at[pl.ds(0, 64)],
                        acc.at[pl.ds(sid * RPT + j * 64, 64)])
        return carry

    lax.fori_loop(0, RPT // 64, zc, 0)
    plsc.subcore_barrier()

    rows = (rows0, rows1)
    sems = (sem0, sem1)

    for half in range(2):
        pltpu.sync_copy(src_hbm.at[wid, pl.ds(half * half_n, half_n)], srci_v)
        pltpu.sync_copy(dst_hbm.at[wid, pl.ds(half * half_n, half_n)], dsti_v)

        pltpu.async_copy(hp_hbm.at[srci_v.at[0]], rows0, sem0)

        def pair(base, carry):
            for b in range(2):
                c = base * 2 + b
                nb = 1 - b

                @pl.when(c + 1 < half_n)
                def _():
                    pltpu.async_copy(hp_hbm.at[srci_v.at[c + 1]],
                                     rows[nb], sems[nb])

                pltpu.make_async_copy(hp_hbm.at[srci_v.at[c]],
                                      rows[b], sems[b]).wait()
                pltpu.sync_copy(rows[b], acc.at[dsti_v.at[c]], add=True)
            return carry

        lax.fori_loop(0, half_n // 2, pair, 0)

    plsc.subcore_barrier()
    pltpu.sync_copy(acc.at[pl.ds(sid * RPT, RPT)],
                    out_hbm.at[cid, pl.ds(sid * RPT, RPT)])


def _dis_from(degT):
    return lax.rsqrt(jnp.sum(degT, axis=1, keepdims=True) + 2.0)


def _pre_body(x_ref, w_ref, degp_ref, out_ref):
    dis = _dis_from(degp_ref[...])
    out_ref[...] = dis * jnp.dot(x_ref[...], w_ref[...],
                                 preferred_element_type=jnp.float32)


def _mid_body(s_ref, hp_ref, degp_ref, b_ref, g_ref, bt_ref, hres_ref,
              wn_ref, h_out_ref, hpn_out_ref):
    dis = _dis_from(degp_ref[...])
    pre = dis * (s_ref[0, :N] + s_ref[1, :N] + 2.0 * hp_ref[...]) + b_ref[...]
    mu = jnp.mean(pre, axis=0, keepdims=True)
    var = jnp.mean((pre - mu) ** 2, axis=0, keepdims=True)
    bn = g_ref[...] * (pre - mu) * lax.rsqrt(var + EPS) + bt_ref[...]
    h_new = jnp.maximum(bn, 0.0) + hres_ref[...]
    h_out_ref[...] = h_new
    hpn_out_ref[...] = dis * jnp.dot(h_new, wn_ref[...],
                                     preferred_element_type=jnp.float32)


NB = 10
BLK = N // NB
NBF = 8
BLKF = NPAD // NBF


def _final_body(s_ref, hp_ref, degp_ref, b_ref, batch_ref, fcw_ref, fcb_ref,
                out_ref):
    i = pl.program_id(0)
    dis = lax.rsqrt(jnp.sum(degp_ref[...], axis=1, keepdims=True) + 2.0)
    h3 = dis * (s_ref[0] + s_ref[1] + 2.0 * hp_ref[...]) + b_ref[...]
    gids = lax.broadcasted_iota(jnp.int32, (G, BLKF), 0)
    onehot = (gids == batch_ref[...]).astype(jnp.float32)
    pooled = jnp.dot(onehot, h3, preferred_element_type=jnp.float32)

    @pl.when(i == 0)
    def _():
        out_ref[...] = jnp.zeros((G, 128), jnp.float32)

    out_ref[...] += pooled

    @pl.when(i == NBF - 1)
    def _():
        res = jnp.dot(out_ref[...], fcw_ref[...],
                      preferred_element_type=jnp.float32) + fcb_ref[...]
        out_ref[...] = jnp.broadcast_to(res, (G, 128))


_pre_call = pl.pallas_call(
    _pre_body,
    grid=(NB,),
    in_specs=[
        pl.BlockSpec((BLK, D), lambda i: (i, 0)),
        pl.BlockSpec((D, D), lambda i: (0, 0)),
        pl.BlockSpec((BLK, NW), lambda i: (i, 0)),
    ],
    out_specs=pl.BlockSpec((BLK, D), lambda i: (i, 0)),
    out_shape=jax.ShapeDtypeStruct((N, D), jnp.float32))

_mid_call = pl.pallas_call(
    _mid_body,
    out_shape=(jax.ShapeDtypeStruct((N, D), jnp.float32),
               jax.ShapeDtypeStruct((N, D), jnp.float32)))

_final_call = pl.pallas_call(
    _final_body,
    grid=(NBF,),
    in_specs=[
        pl.BlockSpec((2, BLKF, D), lambda i: (0, i, 0)),
        pl.BlockSpec((BLKF, D), lambda i: (i, 0)),
        pl.BlockSpec((BLKF, NW), lambda i: (i, 0)),
        pl.BlockSpec((1, D), lambda i: (0, 0)),
        pl.BlockSpec((1, BLKF), lambda i: (0, i)),
        pl.BlockSpec((D, 1), lambda i: (0, 0)),
        pl.BlockSpec((1, 1), lambda i: (0, 0)),
    ],
    out_specs=pl.BlockSpec((G, 128), lambda i: (0, 0)),
    out_shape=jax.ShapeDtypeStruct((G, 128), jnp.float32))


def kernel(x, edge_index, batch, W1, b1, g1, bt1, W2, b2, g2, bt2, W3, b3,
           fcW, fcb):
    src = edge_index[0].reshape(NW, NCH, CHUNK)
    dst = edge_index[1].reshape(NW, NCH, CHUNK)

    degp = _deg_kernel(edge_index[1].reshape(NW, EPW)).T

    b1r = b1.reshape(1, D); g1r = g1.reshape(1, D); bt1r = bt1.reshape(1, D)
    b2r = b2.reshape(1, D); g2r = g2.reshape(1, D); bt2r = bt2.reshape(1, D)
    b3r = b3.reshape(1, D)
    batch_r = batch.reshape(1, N)
    fcb_r = fcb.reshape(1, 1)

    h1p = _pre_call(x, W1, degp)
    s1 = _scatter_kernel(h1p, src, dst)
    h_after1, h2p = _mid_call(s1, h1p, degp, b1r, g1r, bt1r, x, W2)
    s2 = _scatter_kernel(h2p, src, dst)
    h_after2, h3p = _mid_call(s2, h2p, degp, b2r, g2r, bt2r, h_after1, W3)
    s3 = _scatter_kernel(h3p, src, dst)
    h3p_pad = jnp.pad(h3p, ((0, NPAD - N), (0, 0)))
    degp_pad = jnp.pad(degp, ((0, NPAD - N), (0, 0)))
    batch_pad = jnp.pad(batch_r, ((0, 0), (0, NPAD - N)), constant_values=G)
    out = _final_call(s3, h3p_pad, degp_pad, b3r, batch_pad, fcW, fcb_r)
    return out[:, :1]


# speedup vs baseline: 1.1132x; 1.0081x over previous
import functools

import jax
import jax.numpy as jnp
from jax import lax
from jax.experimental import pallas as pl
from jax.experimental.pallas import tpu as pltpu
from jax.experimental.pallas import tpu_sc as plsc

N = 10000
E = 320000
D = 128
G = 8
EPS = 1e-5

NC = 2
NS = 16
NW = NC * NS
EPW = E // NW
CHUNK = 125
NCH = EPW // CHUNK
NSLOT = 2
NSEG = 5
CPS = NCH // NSEG
NPAD = 10240
RPT = NPAD // NS

_mesh = plsc.VectorSubcoreMesh(core_axis_name="c", subcore_axis_name="s")


@functools.partial(
    pl.kernel,
    out_type=jax.ShapeDtypeStruct((NW, N), jnp.float32),
    mesh=_mesh,
    scratch_types=[
        pltpu.VMEM((EPW,), jnp.int32),
        pltpu.VMEM((N,), jnp.float32),
    ],
    compiler_params=pltpu.CompilerParams(needs_layout_passes=False),
)
def _deg_kernel(dst_hbm, out_hbm, dsti_v, hist_v):
    wid = lax.axis_index("s") * NC + lax.axis_index("c")

    zeros16 = jnp.zeros((16,), jnp.float32)

    def zbody(i, carry):
        hist_v[pl.ds(i * 16, 16)] = zeros16
        return carry

    lax.fori_loop(0, N // 16, zbody, 0, unroll=4)

    pltpu.sync_copy(dst_hbm.at[wid], dsti_v)

    ones16 = jnp.ones((16,), jnp.float32)

    def body(i, carry):
        idx = dsti_v[pl.ds(i * 16, 16)]
        plsc.addupdate_scatter(hist_v, [idx], ones16)
        return carry

    lax.fori_loop(0, EPW // 16, body, 0, unroll=4)

    pltpu.sync_copy(hist_v, out_hbm.at[wid])


@functools.partial(
    pl.kernel,
    out_type=jax.ShapeDtypeStruct((NC, NPAD, D), jnp.float32),
    mesh=_mesh,
    scratch_types=[
        pltpu.VMEM_SHARED((NPAD, D), jnp.float32),
        pltpu.VMEM((NCH // 2, CHUNK), jnp.int32),
        pltpu.VMEM((NCH // 2, CHUNK), jnp.int32),
        pltpu.VMEM((CHUNK, D), jnp.float32),
        pltpu.VMEM((CHUNK, D), jnp.float32),
        pltpu.SemaphoreType.DMA,
        pltpu.SemaphoreType.DMA,
    ],
)
def _scatter_kernel(hp_hbm, src_hbm, dst_hbm, out_hbm,
                    acc, srci_v, dsti_v, rows0, rows1, sem0, sem1):
    cid = lax.axis_index("c")
    sid = lax.axis_index("s")
    wid = sid * NC + cid
    half_n = NCH // 2

    zeros16 = jnp.zeros((16,), jnp.float32)

    def zb(i, carry):
        rows0[i // 8, pl.ds((i % 8) * 16, 16)] = zeros16
        return carry

    lax.fori_loop(0, 64 * 8, zb, 0, unroll=8)

    def zc(j, carry):
        pltpu.sync_copy(rows0.at[pl.ds(0, 64)],
                        acc.at[pl.ds(sid * RPT + j * 64, 64)])
        return carry

    lax.fori_loop(0, RPT // 64, zc, 0)
    plsc.subcore_barrier()

    rows = (rows0, rows1)
    sems = (sem0, sem1)

    for half in range(2):
        pltpu.sync_copy(src_hbm.at[wid, pl.ds(half * half_n, half_n)], srci_v)
        pltpu.sync_copy(dst_hbm.at[wid, pl.ds(half * half_n, half_n)], dsti_v)

        pltpu.async_copy(hp_hbm.at[srci_v.at[0]], rows0, sem0)

        def pair(base, carry):
            for b in range(2):
                c = base * 2 + b
                nb = 1 - b

                @pl.when(c + 1 < half_n)
                def _():
                    pltpu.async_copy(hp_hbm.at[srci_v.at[c + 1]],
                                     rows[nb], sems[nb])

                pltpu.make_async_copy(hp_hbm.at[srci_v.at[c]],
                                      rows[b], sems[b]).wait()
                pltpu.sync_copy(rows[b], acc.at[dsti_v.at[c]], add=True)
            return carry

        lax.fori_loop(0, half_n // 2, pair, 0)

    plsc.subcore_barrier()
    pltpu.sync_copy(acc.at[pl.ds(sid * RPT, RPT)],
                    out_hbm.at[cid, pl.ds(sid * RPT, RPT)])


def _dis_from(degT):
    return lax.rsqrt(jnp.sum(degT, axis=1, keepdims=True) + 2.0)


def _pre_body(x_ref, w_ref, degp_ref, out_ref):
    dis = _dis_from(degp_ref[...])
    out_ref[...] = dis * jnp.dot(x_ref[...], w_ref[...],
                                 preferred_element_type=jnp.float32)


def _mid_body(s_ref, hp_ref, degp_ref, b_ref, g_ref, bt_ref, hres_ref,
              wn_ref, h_out_ref, hpn_out_ref):
    dis = _dis_from(degp_ref[...])
    pre = dis * (s_ref[0, :N] + s_ref[1, :N] + 2.0 * hp_ref[...]) + b_ref[...]
    mu = jnp.mean(pre, axis=0, keepdims=True)
    var = jnp.mean((pre - mu) ** 2, axis=0, keepdims=True)
    bn = g_ref[...] * (pre - mu) * lax.rsqrt(var + EPS) + bt_ref[...]
    h_new = jnp.maximum(bn, 0.0) + hres_ref[...]
    h_out_ref[...] = h_new
    hpn_out_ref[...] = dis * jnp.dot(h_new, wn_ref[...],
                                     preferred_element_type=jnp.float32)


NB = 10
BLK = N // NB
NBF = 8
BLKF = NPAD // NBF


def _final_body(s_ref, hp_ref, degp_ref, b_ref, batch_ref, fcw_ref, fcb_ref,
                out_ref):
    dis = _dis_from(degp_ref[...])
    h3 = dis * (s_ref[0, :N] + s_ref[1, :N] + 2.0 * hp_ref[...]) + b_ref[...]
    gids = lax.broadcasted_iota(jnp.int32, (G, N), 0)
    onehot = (gids == batch_ref[...]).astype(jnp.float32)
    pooled = jnp.dot(onehot, h3, preferred_element_type=jnp.float32)
    res = jnp.dot(pooled, fcw_ref[...],
                  preferred_element_type=jnp.float32) + fcb_ref[...]
    out_ref[...] = jnp.broadcast_to(res, (G, 128))


_pre_call = pl.pallas_call(
    _pre_body,
    grid=(NB,),
    in_specs=[
        pl.BlockSpec((BLK, D), lambda i: (i, 0)),
        pl.BlockSpec((D, D), lambda i: (0, 0)),
        pl.BlockSpec((BLK, NW), lambda i: (i, 0)),
    ],
    out_specs=pl.BlockSpec((BLK, D), lambda i: (i, 0)),
    out_shape=jax.ShapeDtypeStruct((N, D), jnp.float32))

_mid_call = pl.pallas_call(
    _mid_body,
    out_shape=(jax.ShapeDtypeStruct((N, D), jnp.float32),
               jax.ShapeDtypeStruct((N, D), jnp.float32)))

_final_call = pl.pallas_call(
    _final_body, out_shape=jax.ShapeDtypeStruct((G, 128), jnp.float32))


def kernel(x, edge_index, batch, W1, b1, g1, bt1, W2, b2, g2, bt2, W3, b3,
           fcW, fcb):
    src = edge_index[0].reshape(NW, NCH, CHUNK)
    dst = edge_index[1].reshape(NW, NCH, CHUNK)

    degp = _deg_kernel(edge_index[1].reshape(NW, EPW)).T

    b1r = b1.reshape(1, D); g1r = g1.reshape(1, D); bt1r = bt1.reshape(1, D)
    b2r = b2.reshape(1, D); g2r = g2.reshape(1, D); bt2r = bt2.reshape(1, D)
    b3r = b3.reshape(1, D)
    batch_r = batch.reshape(1, N)
    fcb_r = fcb.reshape(1, 1)

    h1p = _pre_call(x, W1, degp)
    s1 = _scatter_kernel(h1p, src, dst)
    h_after1, h2p = _mid_call(s1, h1p, degp, b1r, g1r, bt1r, x, W2)
    s2 = _scatter_kernel(h2p, src, dst)
    h_after2, h3p = _mid_call(s2, h2p, degp, b2r, g2r, bt2r, h_after1, W3)
    s3 = _scatter_kernel(h3p, src, dst)
    out = _final_call(s3, h3p, degp, b3r, batch_r, fcW, fcb_r)
    return out[:, :1]


# speedup vs baseline: 1.1229x; 1.0087x over previous
import functools

import jax
import jax.numpy as jnp
from jax import lax
from jax.experimental import pallas as pl
from jax.experimental.pallas import tpu as pltpu
from jax.experimental.pallas import tpu_sc as plsc

N = 10000
E = 320000
D = 128
G = 8
EPS = 1e-5

NC = 2
NS = 16
NW = NC * NS
EPW = E // NW
CHUNK = 125
NCH = EPW // CHUNK
NPAD = 10240
RPT = NPAD // NS

_mesh = plsc.VectorSubcoreMesh(core_axis_name="c", subcore_axis_name="s")


@functools.partial(
    pl.kernel,
    out_type=jax.ShapeDtypeStruct((NW, N), jnp.float32),
    mesh=_mesh,
    scratch_types=[
        pltpu.VMEM((EPW,), jnp.int32),
        pltpu.VMEM((N,), jnp.float32),
    ],
    compiler_params=pltpu.CompilerParams(needs_layout_passes=False),
)
def _deg_kernel(dst_hbm, out_hbm, dsti_v, hist_v):
    wid = lax.axis_index("s") * NC + lax.axis_index("c")

    zeros16 = jnp.zeros((16,), jnp.float32)

    def zbody(i, carry):
        hist_v[pl.ds(i * 16, 16)] = zeros16
        return carry

    lax.fori_loop(0, N // 16, zbody, 0, unroll=4)

    pltpu.sync_copy(dst_hbm.at[wid], dsti_v)

    ones16 = jnp.ones((16,), jnp.float32)

    def body(i, carry):
        idx = dsti_v[pl.ds(i * 16, 16)]
        plsc.addupdate_scatter(hist_v, [idx], ones16)
        return carry

    lax.fori_loop(0, EPW // 16, body, 0, unroll=4)

    pltpu.sync_copy(hist_v, out_hbm.at[wid])


@functools.partial(
    pl.kernel,
    out_type=jax.ShapeDtypeStruct((NC, NPAD, D), jnp.float32),
    mesh=_mesh,
    scratch_types=[
        pltpu.VMEM_SHARED((NPAD, D), jnp.float32),
        pltpu.VMEM((NCH // 2, CHUNK), jnp.int32),
        pltpu.VMEM((NCH // 2, CHUNK), jnp.int32),
        pltpu.VMEM((CHUNK, D), jnp.float32),
        pltpu.VMEM((CHUNK, D), jnp.float32),
        pltpu.SemaphoreType.DMA,
        pltpu.SemaphoreType.DMA,
    ],
)
def _scatter_kernel(hp_hbm, src_hbm, dst_hbm, out_hbm,
                    acc, srci_v, dsti_v, rows0, rows1, sem0, sem1):
    cid = lax.axis_index("c")
    sid = lax.axis_index("s")
    wid = sid * NC + cid
    half_n = NCH // 2

    zeros16 = jnp.zeros((16,), jnp.float32)

    def zb(i, carry):
        rows0[i // 8, pl.ds((i % 8) * 16, 16)] = zeros16
        return carry

    lax.fori_loop(0, 64 * 8, zb, 0, unroll=8)

    def zc(j, carry):
        pltpu.sync_copy(rows0.at[pl.ds(0, 64)],
                        acc.at[pl.ds(sid * RPT + j * 64, 64)])
        return carry

    lax.fori_loop(0, RPT // 64, zc, 0)
    plsc.subcore_barrier()

    rows = (rows0, rows1)
    sems = (sem0, sem1)

    for half in range(2):
        pltpu.sync_copy(src_hbm.at[wid, pl.ds(half * half_n, half_n)], srci_v)
        pltpu.sync_copy(dst_hbm.at[wid, pl.ds(half * half_n, half_n)], dsti_v)

        pltpu.async_copy(hp_hbm.at[srci_v.at[0]], rows0, sem0)

        def pair(base, carry):
            for b in range(2):
                c = base * 2 + b
                nb = 1 - b

                @pl.when(c + 1 < half_n)
                def _():
                    pltpu.async_copy(hp_hbm.at[srci_v.at[c + 1]],
                                     rows[nb], sems[nb])

                pltpu.make_async_copy(hp_hbm.at[srci_v.at[c]],
                                      rows[b], sems[b]).wait()
                pltpu.sync_copy(rows[b], acc.at[dsti_v.at[c]], add=True)
            return carry

        lax.fori_loop(0, half_n // 2, pair, 0)

    plsc.subcore_barrier()
    pltpu.sync_copy(acc.at[pl.ds(sid * RPT, RPT)],
                    out_hbm.at[cid, pl.ds(sid * RPT, RPT)])


def _dis_from(degT):
    return lax.rsqrt(jnp.sum(degT, axis=1, keepdims=True) + 2.0)


def _pre_body(x_ref, w_ref, degp_ref, out_ref):
    dis = _dis_from(degp_ref[...])
    out_ref[...] = dis * jnp.dot(x_ref[...], w_ref[...],
                                 preferred_element_type=jnp.float32)


def _mid_body(s_ref, hp_ref, degp_ref, b_ref, g_ref, bt_ref, hres_ref,
              wn_ref, h_out_ref, hpn_out_ref):
    dis = _dis_from(degp_ref[...])
    pre = dis * (s_ref[0, :N] + s_ref[1, :N] + 2.0 * hp_ref[...]) + b_ref[...]
    mu = jnp.mean(pre, axis=0, keepdims=True)
    var = jnp.mean((pre - mu) ** 2, axis=0, keepdims=True)
    bn = g_ref[...] * (pre - mu) * lax.rsqrt(var + EPS) + bt_ref[...]
    h_new = jnp.maximum(bn, 0.0) + hres_ref[...]
    h_out_ref[...] = h_new
    hpn_out_ref[...] = dis * jnp.dot(h_new, wn_ref[...],
                                     preferred_element_type=jnp.float32)


def _final_body(s_ref, hp_ref, degp_ref, b_ref, batch_ref, fcw_ref, fcb_ref,
                out_ref):
    dis = _dis_from(degp_ref[...])
    h3 = dis * (s_ref[0, :N] + s_ref[1, :N] + 2.0 * hp_ref[...]) + b_ref[...]
    gids = lax.broadcasted_iota(jnp.int32, (G, N), 0)
    onehot = (gids == batch_ref[...]).astype(jnp.float32)
    pooled = jnp.dot(onehot, h3, preferred_element_type=jnp.float32)
    res = jnp.dot(pooled, fcw_ref[...],
                  preferred_element_type=jnp.float32) + fcb_ref[...]
    out_ref[...] = jnp.broadcast_to(res, (G, 128))


_pre_call = pl.pallas_call(
    _pre_body, out_shape=jax.ShapeDtypeStruct((N, D), jnp.float32))

_mid_call = pl.pallas_call(
    _mid_body,
    out_shape=(jax.ShapeDtypeStruct((N, D), jnp.float32),
               jax.ShapeDtypeStruct((N, D), jnp.float32)))

_final_call = pl.pallas_call(
    _final_body, out_shape=jax.ShapeDtypeStruct((G, 128), jnp.float32))


def kernel(x, edge_index, batch, W1, b1, g1, bt1, W2, b2, g2, bt2, W3, b3,
           fcW, fcb):
    src = edge_index[0].reshape(NW, NCH, CHUNK)
    dst = edge_index[1].reshape(NW, NCH, CHUNK)

    degp = _deg_kernel(edge_index[1].reshape(NW, EPW)).T

    b1r = b1.reshape(1, D); g1r = g1.reshape(1, D); bt1r = bt1.reshape(1, D)
    b2r = b2.reshape(1, D); g2r = g2.reshape(1, D); bt2r = bt2.reshape(1, D)
    b3r = b3.reshape(1, D)
    batch_r = batch.reshape(1, N)
    fcb_r = fcb.reshape(1, 1)

    h1p = _pre_call(x, W1, degp)
    s1 = _scatter_kernel(h1p, src, dst)
    h_after1, h2p = _mid_call(s1, h1p, degp, b1r, g1r, bt1r, x, W2)
    s2 = _scatter_kernel(h2p, src, dst)
    h_after2, h3p = _mid_call(s2, h2p, degp, b2r, g2r, bt2r, h_after1, W3)
    s3 = _scatter_kernel(h3p, src, dst)
    out = _final_call(s3, h3p, degp, b3r, batch_r, fcW, fcb_r)
    return out[:, :1]
